# Initial kernel scaffold; baseline (speedup 1.0000x reference)
import functools

import jax
import jax.numpy as jnp
from jax import lax
from jax.experimental import pallas as pl
from jax.experimental.pallas import tpu as pltpu
from jax.experimental.pallas import tpu_sc as plsc

_NC = 2
_NS = 16
_NW = _NC * _NS
_L = 16
_CHUNK = 128
_DW = 144


def _encode_body(x_ref, we_ref, be_ref, ws_ref, bs_ref, wm_ref, bm_ref,
                 h_ref, sig_ref, msg_ref):
    x = x_ref[...]
    h = jnp.dot(x, we_ref[...], preferred_element_type=jnp.float32) + be_ref[...]
    s = jnp.dot(h, ws_ref[...], preferred_element_type=jnp.float32) + bs_ref[...]
    nrm = jnp.sqrt(jnp.sum(s * s, axis=-1, keepdims=True))
    sig_ref[...] = s / jnp.maximum(nrm, 1e-12)
    msg_ref[...] = jnp.maximum(
        jnp.dot(h, wm_ref[...], preferred_element_type=jnp.float32) + bm_ref[...],
        0.0)
    h_ref[...] = h


def _encode(x, W_enc, b_enc, W_sig, b_sig, W_msg, b_msg, block_rows):
    n, d_in = x.shape
    d_h = W_enc.shape[1]
    d_sig = W_sig.shape[1]
    grid = (n // block_rows,)
    full = lambda shape: pl.BlockSpec(shape, lambda i: (0, 0))
    return pl.pallas_call(
        _encode_body,
        grid=grid,
        in_specs=[
            pl.BlockSpec((block_rows, d_in), lambda i: (i, 0)),
            full((d_in, d_h)), full((1, d_h)),
            full((d_h, d_sig)), full((1, d_sig)),
            full((d_h, d_h)), full((1, d_h)),
        ],
        out_specs=[
            pl.BlockSpec((block_rows, d_h), lambda i: (i, 0)),
            pl.BlockSpec((block_rows, d_sig), lambda i: (i, 0)),
            pl.BlockSpec((block_rows, d_h), lambda i: (i, 0)),
        ],
        out_shape=[
            jax.ShapeDtypeStruct((n, d_h), jnp.float32),
            jax.ShapeDtypeStruct((n, d_sig), jnp.float32),
            jax.ShapeDtypeStruct((n, d_h), jnp.float32),
        ],
    )(x, W_enc, b_enc.reshape(1, -1), W_sig, b_sig.reshape(1, -1),
      W_msg, b_msg.reshape(1, -1))


def _sc_edge_kernel(n_nodes, n_edges, cpw, d_sig, d_h):
    rows_per_tile = n_nodes // _NS
    zrows = 125
    mesh = plsc.VectorSubcoreMesh(core_axis_name="c", subcore_axis_name="s")

    def body(sig_hbm, msg_hbm, src_hbm, dst_hbm, out_hbm,
             src_v, dst_v, ssrc_v, sdst_v, msg_v, scaled_v, ex_v, zero_v,
             p_sh, sem0, sem1, sem2):
        cid = lax.axis_index("c")
        sid = lax.axis_index("s")
        wid = sid * _NC + cid
        lane = lax.broadcasted_iota(jnp.int32, (_L,), 0)

        def zfill(r, _):
            for q in range(_DW // _L):
                zero_v[r, pl.ds(q * _L, _L)] = jnp.zeros((_L,), jnp.float32)
            return 0
        lax.fori_loop(0, zrows, zfill, 0, unroll=False)
        for k in range(rows_per_tile // zrows):
            pltpu.sync_copy(
                zero_v, p_sh.at[pl.ds(sid * rows_per_tile + k * zrows, zrows)])
        plsc.subcore_barrier()

        pltpu.sync_copy(src_hbm.at[wid], src_v)
        pltpu.sync_copy(dst_hbm.at[wid], dst_v)

        ebase = wid * (cpw * _CHUNK)

        def chunk_body(j, _):
            cp0 = pltpu.async_copy(sig_hbm.at[src_v.at[j]], ssrc_v, sem0)
            cp1 = pltpu.async_copy(sig_hbm.at[dst_v.at[j]], sdst_v, sem1)
            cp2 = pltpu.async_copy(msg_hbm.at[dst_v.at[j]], msg_v, sem2)
            cp0.wait()
            cp1.wait()

            for g in range(_CHUNK // _L):
                rows = g * _L + lane
                acc = jnp.zeros((_L,), jnp.float32)
                for f in range(d_sig):
                    col = jnp.full((_L,), f, jnp.int32)
                    a = plsc.load_gather(ssrc_v, [rows, col])
                    b = plsc.load_gather(sdst_v, [rows, col])
                    acc = acc + a * b
                eid = ebase + j * _CHUNK + g * _L + lane
                ex = jnp.where(eid < n_edges, jnp.exp(acc), 0.0)
                ex_v[pl.ds(g * _L, _L)] = ex

            cp2.wait()

            def scale_row(r, _):
                spl = plsc.load_gather(ex_v, [jnp.full((_L,), r, jnp.int32)])
                for q in range(d_h // _L):
                    scaled_v[r, pl.ds(q * _L, _L)] = (
                        msg_v[r, pl.ds(q * _L, _L)] * spl)
                scaled_v[r, pl.ds(d_h, _L)] = jnp.where(
                    lane == 0, spl, jnp.zeros((_L,), jnp.float32))
                return 0
            lax.fori_loop(0, _CHUNK, scale_row, 0, unroll=False)

            pltpu.sync_copy(scaled_v, p_sh.at[src_v.at[j]], add=True)
            return 0

        lax.fori_loop(0, cpw, chunk_body, 0, unroll=False)
        plsc.subcore_barrier()

        pltpu.sync_copy(
            p_sh.at[pl.ds(sid * rows_per_tile, rows_per_tile)],
            out_hbm.at[cid].at[pl.ds(sid * rows_per_tile, rows_per_tile)])

    return pl.kernel(
        body,
        out_type=jax.ShapeDtypeStruct((_NC, n_nodes, _DW), jnp.float32),
        mesh=mesh,
        scratch_types=[
            pltpu.VMEM((cpw, _CHUNK), jnp.int32),
            pltpu.VMEM((cpw, _CHUNK), jnp.int32),
            pltpu.VMEM((_CHUNK, d_sig), jnp.float32),
            pltpu.VMEM((_CHUNK, d_sig), jnp.float32),
            pltpu.VMEM((_CHUNK, d_h), jnp.float32),
            pltpu.VMEM((_CHUNK, _DW), jnp.float32),
            pltpu.VMEM((_CHUNK,), jnp.float32),
            pltpu.VMEM((125, _DW), jnp.float32),
            pltpu.VMEM_SHARED((n_nodes, _DW), jnp.float32),
            pltpu.SemaphoreType.DMA,
            pltpu.SemaphoreType.DMA,
            pltpu.SemaphoreType.DMA,
        ],
    )


def _combine_body(h_ref, p0_ref, p1_ref, wh_ref, wc_ref, ba_ref,
                  wd_ref, bd_ref, out_ref):
    p = p0_ref[...] + p1_ref[...]
    denom = jnp.maximum(p[:, 128:129], 1e-30)
    comm = p[:, :128] / denom
    h = h_ref[...]
    combined = jnp.maximum(
        jnp.dot(h, wh_ref[...], preferred_element_type=jnp.float32)
        + jnp.dot(comm, wc_ref[...], preferred_element_type=jnp.float32)
        + ba_ref[...], 0.0)
    out_ref[...] = (
        jnp.dot(combined, wd_ref[...], preferred_element_type=jnp.float32)
        + bd_ref[...])


def _combine(h, p0, p1, W_agg, b_agg, W_dec, b_dec, block_rows):
    n, d_h = h.shape
    d_out = W_dec.shape[1]
    grid = (n // block_rows,)
    full = lambda shape: pl.BlockSpec(shape, lambda i: (0, 0))
    return pl.pallas_call(
        _combine_body,
        grid=grid,
        in_specs=[
            pl.BlockSpec((block_rows, d_h), lambda i: (i, 0)),
            pl.BlockSpec((block_rows, _DW), lambda i: (i, 0)),
            pl.BlockSpec((block_rows, _DW), lambda i: (i, 0)),
            full((d_h, d_h)), full((d_h, d_h)), full((1, d_h)),
            full((d_h, d_out)), full((1, d_out)),
        ],
        out_specs=pl.BlockSpec((block_rows, d_out), lambda i: (i, 0)),
        out_shape=jax.ShapeDtypeStruct((n, d_out), jnp.float32),
    )(h, p0, p1, W_agg[:d_h], W_agg[d_h:], b_agg.reshape(1, -1),
      W_dec, b_dec.reshape(1, -1))


@jax.jit
def kernel(x, edge_index, W_enc, b_enc, W_sig, b_sig, W_msg, b_msg,
           W_agg, b_agg, W_dec, b_dec):
    n = x.shape[0]
    e = edge_index.shape[1]
    d_sig = W_sig.shape[1]
    d_h = W_enc.shape[1]

    h, sig, msg = _encode(x, W_enc, b_enc, W_sig, b_sig, W_msg, b_msg,
                          block_rows=500)

    slab = _CHUNK * _NW
    cpw = (e + slab - 1) // slab
    e_pad = cpw * slab
    src = edge_index[0].astype(jnp.int32)
    dst = edge_index[1].astype(jnp.int32)
    pad = e_pad - e
    if pad:
        src = jnp.concatenate([src, jnp.zeros((pad,), jnp.int32)])
        dst = jnp.concatenate([dst, jnp.zeros((pad,), jnp.int32)])
    src3 = src.reshape(_NW, cpw, _CHUNK)
    dst3 = dst.reshape(_NW, cpw, _CHUNK)

    p = _sc_edge_kernel(n, e, cpw, d_sig, d_h)(sig, msg, src3, dst3)

    return _combine(h, p[0], p[1], W_agg, b_agg, W_dec, b_dec, block_rows=500)


# speedup vs baseline: 7.6074x; 7.6074x over previous
import jax
import jax.numpy as jnp
from jax import lax
from jax.experimental import pallas as pl
from jax.experimental.pallas import tpu as pltpu
from jax.experimental.pallas import tpu_sc as plsc

_NC = 2
_NS = 16
_L = 16
_CHUNK = 128
_MH = 64
_DW = _MH + _L


def _encode_body(x_ref, we_ref, be_ref, ws_ref, bs_ref, wm_ref, bm_ref,
                 h_ref, sig_ref, msg0_ref, msg1_ref):
    x = x_ref[...]
    h = jnp.dot(x, we_ref[...], preferred_element_type=jnp.float32) + be_ref[...]
    s = jnp.dot(h, ws_ref[...], preferred_element_type=jnp.float32) + bs_ref[...]
    nrm = jnp.sqrt(jnp.sum(s * s, axis=-1, keepdims=True))
    sig_ref[...] = s / jnp.maximum(nrm, 1e-12)
    msg = jnp.maximum(
        jnp.dot(h, wm_ref[...], preferred_element_type=jnp.float32) + bm_ref[...],
        0.0)
    msg0_ref[...] = msg[:, :_MH]
    msg1_ref[...] = msg[:, _MH:]
    h_ref[...] = h


def _encode(x, W_enc, b_enc, W_sig, b_sig, W_msg, b_msg, block_rows):
    n, d_in = x.shape
    d_h = W_enc.shape[1]
    d_sig = W_sig.shape[1]
    grid = (n // block_rows,)
    full = lambda shape: pl.BlockSpec(shape, lambda i: (0, 0))
    return pl.pallas_call(
        _encode_body,
        grid=grid,
        in_specs=[
            pl.BlockSpec((block_rows, d_in), lambda i: (i, 0)),
            full((d_in, d_h)), full((1, d_h)),
            full((d_h, d_sig)), full((1, d_sig)),
            full((d_h, d_h)), full((1, d_h)),
        ],
        out_specs=[
            pl.BlockSpec((block_rows, d_h), lambda i: (i, 0)),
            pl.BlockSpec((block_rows, d_sig), lambda i: (i, 0)),
            pl.BlockSpec((block_rows, _MH), lambda i: (i, 0)),
            pl.BlockSpec((block_rows, _MH), lambda i: (i, 0)),
        ],
        out_shape=[
            jax.ShapeDtypeStruct((n, d_h), jnp.float32),
            jax.ShapeDtypeStruct((n, d_sig), jnp.float32),
            jax.ShapeDtypeStruct((n, _MH), jnp.float32),
            jax.ShapeDtypeStruct((n, _MH), jnp.float32),
        ],
    )(x, W_enc, b_enc.reshape(1, -1), W_sig, b_sig.reshape(1, -1),
      W_msg, b_msg.reshape(1, -1))


def _sc_edge_kernel(n_nodes, n_edges, cpw, d_sig):
    zrows = 200
    nblocks = n_nodes // zrows
    mesh = plsc.VectorSubcoreMesh(core_axis_name="c", subcore_axis_name="s")

    def body(sig_hbm, msgh_hbm, src_hbm, dst_hbm, out_hbm,
             src_v, dst_v, midx_v, ssrc_v, sdst_v, msg_v, scaled_v, ex_v,
             zero_v, p_sh, sem0, sem1, sem2):
        cid = lax.axis_index("c")
        sid = lax.axis_index("s")
        lane = lax.broadcasted_iota(jnp.int32, (_L,), 0)

        def zfill(r, _):
            for q in range(_DW // _L):
                zero_v[r, pl.ds(q * _L, _L)] = jnp.zeros((_L,), jnp.float32)
            return 0
        lax.fori_loop(0, zrows, zfill, 0, unroll=False)
        for k in range((nblocks + _NS - 1) // _NS):
            b = sid + _NS * k
            @pl.when(b < nblocks)
            def _():
                pltpu.sync_copy(zero_v, p_sh.at[pl.ds(b * zrows, zrows)])
        plsc.subcore_barrier()

        pltpu.sync_copy(src_hbm.at[sid], src_v)
        pltpu.sync_copy(dst_hbm.at[sid], dst_v)
        moff = cid * n_nodes

        def shift_row(r, _):
            for q in range(_CHUNK // _L):
                midx_v[r, pl.ds(q * _L, _L)] = (
                    dst_v[r, pl.ds(q * _L, _L)] + moff)
            return 0
        lax.fori_loop(0, cpw, shift_row, 0, unroll=False)

        ebase = sid * (cpw * _CHUNK)

        def chunk_body(j, _):
            cp0 = pltpu.async_copy(sig_hbm.at[src_v.at[j]], ssrc_v, sem0)
            cp1 = pltpu.async_copy(sig_hbm.at[dst_v.at[j]], sdst_v, sem1)
            cp2 = pltpu.async_copy(msgh_hbm.at[midx_v.at[j]], msg_v, sem2)
            cp0.wait()
            cp1.wait()

            for g in range(_CHUNK // _L):
                rows = g * _L + lane
                acc = jnp.zeros((_L,), jnp.float32)
                for f in range(d_sig):
                    col = jnp.full((_L,), f, jnp.int32)
                    a = plsc.load_gather(ssrc_v, [rows, col])
                    b = plsc.load_gather(sdst_v, [rows, col])
                    acc = acc + a * b
                eid = ebase + j * _CHUNK + g * _L + lane
                ex = jnp.where(eid < n_edges, jnp.exp(acc), 0.0)
                ex_v[pl.ds(g * _L, _L)] = ex

            cp2.wait()

            def scale_row(r, _):
                spl = plsc.load_gather(ex_v, [jnp.full((_L,), r, jnp.int32)])
                for q in range(_MH // _L):
                    scaled_v[r, pl.ds(q * _L, _L)] = (
                        msg_v[r, pl.ds(q * _L, _L)] * spl)
                scaled_v[r, pl.ds(_MH, _L)] = jnp.where(
                    lane == 0, spl, jnp.zeros((_L,), jnp.float32))
                return 0
            lax.fori_loop(0, _CHUNK, scale_row, 0, unroll=False)

            pltpu.sync_copy(scaled_v, p_sh.at[src_v.at[j]], add=True)
            return 0

        lax.fori_loop(0, cpw, chunk_body, 0, unroll=False)
        plsc.subcore_barrier()

        for k in range((nblocks + _NS - 1) // _NS):
            b = sid + _NS * k
            @pl.when(b < nblocks)
            def _():
                pltpu.sync_copy(
                    p_sh.at[pl.ds(b * zrows, zrows)],
                    out_hbm.at[cid].at[pl.ds(b * zrows, zrows)])

    return pl.kernel(
        body,
        out_type=jax.ShapeDtypeStruct((_NC, n_nodes, _DW), jnp.float32),
        mesh=mesh,
        scratch_types=[
            pltpu.VMEM((cpw, _CHUNK), jnp.int32),
            pltpu.VMEM((cpw, _CHUNK), jnp.int32),
            pltpu.VMEM((cpw, _CHUNK), jnp.int32),
            pltpu.VMEM((_CHUNK, d_sig), jnp.float32),
            pltpu.VMEM((_CHUNK, d_sig), jnp.float32),
            pltpu.VMEM((_CHUNK, _MH), jnp.float32),
            pltpu.VMEM((_CHUNK, _DW), jnp.float32),
            pltpu.VMEM((_CHUNK,), jnp.float32),
            pltpu.VMEM((200, _DW), jnp.float32),
            pltpu.VMEM_SHARED((n_nodes, _DW), jnp.float32),
            pltpu.SemaphoreType.DMA,
            pltpu.SemaphoreType.DMA,
            pltpu.SemaphoreType.DMA,
        ],
        compiler_params=pltpu.CompilerParams(
            needs_layout_passes=False, use_tc_tiling_on_sc=False),
    )


def _combine_body(h_ref, p0_ref, p1_ref, wh_ref, wc0_ref, wc1_ref, ba_ref,
                  wd_ref, bd_ref, out_ref):
    p0 = p0_ref[...]
    p1 = p1_ref[...]
    comm0 = p0[:, :_MH] / jnp.maximum(p0[:, _MH:_MH + 1], 1e-30)
    comm1 = p1[:, :_MH] / jnp.maximum(p1[:, _MH:_MH + 1], 1e-30)
    combined = jnp.maximum(
        jnp.dot(h_ref[...], wh_ref[...], preferred_element_type=jnp.float32)
        + jnp.dot(comm0, wc0_ref[...], preferred_element_type=jnp.float32)
        + jnp.dot(comm1, wc1_ref[...], preferred_element_type=jnp.float32)
        + ba_ref[...], 0.0)
    out_ref[...] = (
        jnp.dot(combined, wd_ref[...], preferred_element_type=jnp.float32)
        + bd_ref[...])


def _combine(h, p0, p1, W_agg, b_agg, W_dec, b_dec, block_rows):
    n, d_h = h.shape
    d_out = W_dec.shape[1]
    grid = (n // block_rows,)
    full = lambda shape: pl.BlockSpec(shape, lambda i: (0, 0))
    return pl.pallas_call(
        _combine_body,
        grid=grid,
        in_specs=[
            pl.BlockSpec((block_rows, d_h), lambda i: (i, 0)),
            pl.BlockSpec((block_rows, _DW), lambda i: (i, 0)),
            pl.BlockSpec((block_rows, _DW), lambda i: (i, 0)),
            full((d_h, d_h)), full((_MH, d_h)), full((_MH, d_h)),
            full((1, d_h)),
            full((d_h, d_out)), full((1, d_out)),
        ],
        out_specs=pl.BlockSpec((block_rows, d_out), lambda i: (i, 0)),
        out_shape=jax.ShapeDtypeStruct((n, d_out), jnp.float32),
    )(h, p0, p1, W_agg[:d_h], W_agg[d_h:d_h + _MH], W_agg[d_h + _MH:],
      b_agg.reshape(1, -1), W_dec, b_dec.reshape(1, -1))


@jax.jit
def kernel(x, edge_index, W_enc, b_enc, W_sig, b_sig, W_msg, b_msg,
           W_agg, b_agg, W_dec, b_dec):
    n = x.shape[0]
    e = edge_index.shape[1]
    d_sig = W_sig.shape[1]

    h, sig, msg0, msg1 = _encode(x, W_enc, b_enc, W_sig, b_sig, W_msg, b_msg,
                                 block_rows=400)
    msgh = jnp.concatenate([msg0, msg1], axis=0)

    slab = _CHUNK * _NS
    cpw = (e + slab - 1) // slab
    e_pad = cpw * slab
    src = edge_index[0].astype(jnp.int32)
    dst = edge_index[1].astype(jnp.int32)
    pad = e_pad - e
    if pad:
        src = jnp.concatenate([src, jnp.zeros((pad,), jnp.int32)])
        dst = jnp.concatenate([dst, jnp.zeros((pad,), jnp.int32)])
    src3 = src.reshape(_NS, cpw, _CHUNK)
    dst3 = dst.reshape(_NS, cpw, _CHUNK)

    p = _sc_edge_kernel(n, e, cpw, d_sig)(sig, msgh, src3, dst3)

    return _combine(h, p[0], p[1], W_agg, b_agg, W_dec, b_dec, block_rows=400)


# speedup vs baseline: 9.6678x; 1.2709x over previous
import jax
import jax.numpy as jnp
from jax import lax
from jax.experimental import pallas as pl
from jax.experimental.pallas import tpu as pltpu
from jax.experimental.pallas import tpu_sc as plsc

_NC = 2
_NS = 16
_L = 16
_CHUNK = 128
_MH = 64
_DW = _MH + _L


def _encode_body(x_ref, we_ref, be_ref, ws_ref, bs_ref, wm_ref, bm_ref,
                 h_ref, sig_ref, msg0_ref, msg1_ref):
    x = x_ref[...]
    h = jnp.dot(x, we_ref[...], preferred_element_type=jnp.float32) + be_ref[...]
    s = jnp.dot(h, ws_ref[...], preferred_element_type=jnp.float32) + bs_ref[...]
    nrm = jnp.sqrt(jnp.sum(s * s, axis=-1, keepdims=True))
    sig_ref[...] = s / jnp.maximum(nrm, 1e-12)
    msg = jnp.maximum(
        jnp.dot(h, wm_ref[...], preferred_element_type=jnp.float32) + bm_ref[...],
        0.0)
    msg0_ref[...] = msg[:, :_MH]
    msg1_ref[...] = msg[:, _MH:]
    h_ref[...] = h


def _encode(x, W_enc, b_enc, W_sig, b_sig, W_msg, b_msg, block_rows):
    n, d_in = x.shape
    d_h = W_enc.shape[1]
    d_sig = W_sig.shape[1]
    grid = (n // block_rows,)
    full = lambda shape: pl.BlockSpec(shape, lambda i: (0, 0))
    return pl.pallas_call(
        _encode_body,
        grid=grid,
        in_specs=[
            pl.BlockSpec((block_rows, d_in), lambda i: (i, 0)),
            full((d_in, d_h)), full((1, d_h)),
            full((d_h, d_sig)), full((1, d_sig)),
            full((d_h, d_h)), full((1, d_h)),
        ],
        out_specs=[
            pl.BlockSpec((block_rows, d_h), lambda i: (i, 0)),
            pl.BlockSpec((block_rows, d_sig), lambda i: (i, 0)),
            pl.BlockSpec((block_rows, _MH), lambda i: (i, 0)),
            pl.BlockSpec((block_rows, _MH), lambda i: (i, 0)),
        ],
        out_shape=[
            jax.ShapeDtypeStruct((n, d_h), jnp.float32),
            jax.ShapeDtypeStruct((n, d_sig), jnp.float32),
            jax.ShapeDtypeStruct((n, _MH), jnp.float32),
            jax.ShapeDtypeStruct((n, _MH), jnp.float32),
        ],
    )(x, W_enc, b_enc.reshape(1, -1), W_sig, b_sig.reshape(1, -1),
      W_msg, b_msg.reshape(1, -1))


def _sc_edge_kernel(n_nodes, n_edges, cpw, d_sig):
    zrows = 200
    nblocks = n_nodes // zrows
    mesh = plsc.VectorSubcoreMesh(core_axis_name="c", subcore_axis_name="s")

    def body(sig_hbm, msgh_hbm, src_hbm, dst_hbm, out_hbm,
             src_v, dst_v, midx_v, ssrc0, sdst0, msg0, msg1,
             scaled0, ex_v, zero_v, p_sh,
             ga0, gb0, gc0, gc1):
        cid = lax.axis_index("c")
        sid = lax.axis_index("s")
        lane = lax.broadcasted_iota(jnp.int32, (_L,), 0)
        msgb = (msg0, msg1)
        gc = (gc0, gc1)

        def zfill(r, _):
            for q in range(_DW // _L):
                zero_v[r, pl.ds(q * _L, _L)] = jnp.zeros((_L,), jnp.float32)
            return 0
        lax.fori_loop(0, zrows, zfill, 0, unroll=False)
        for k in range((nblocks + _NS - 1) // _NS):
            b = sid + _NS * k
            @pl.when(b < nblocks)
            def _():
                pltpu.sync_copy(zero_v, p_sh.at[pl.ds(b * zrows, zrows)])
        plsc.subcore_barrier()

        pltpu.sync_copy(src_hbm.at[sid], src_v)
        pltpu.sync_copy(dst_hbm.at[sid], dst_v)
        moff = cid * n_nodes

        def shift_row(r, _):
            for q in range(_CHUNK // _L):
                midx_v[r, pl.ds(q * _L, _L)] = (
                    dst_v[r, pl.ds(q * _L, _L)] + moff)
            return 0
        lax.fori_loop(0, cpw, shift_row, 0, unroll=False)

        ebase = sid * (cpw * _CHUNK)

        def issue_sig_gathers(j):
            pltpu.async_copy(sig_hbm.at[src_v.at[j]], ssrc0, ga0)
            pltpu.async_copy(sig_hbm.at[dst_v.at[j]], sdst0, gb0)

        def issue_msg_gather(j, b):
            pltpu.async_copy(msgh_hbm.at[midx_v.at[j]], msgb[b], gc[b])

        issue_sig_gathers(0)
        issue_msg_gather(0, 0)
        issue_msg_gather(1, 1)

        def do_chunk(j, b):
            pltpu.make_async_copy(sig_hbm.at[src_v.at[j]], ssrc0,
                                  ga0).wait()
            pltpu.make_async_copy(sig_hbm.at[dst_v.at[j]], sdst0,
                                  gb0).wait()

            for g in range(_CHUNK // _L):
                rows = g * _L + lane
                acc = jnp.zeros((_L,), jnp.float32)
                for f in range(d_sig):
                    col = jnp.full((_L,), f, jnp.int32)
                    a = plsc.load_gather(ssrc0, [rows, col])
                    bb = plsc.load_gather(sdst0, [rows, col])
                    acc = acc + a * bb
                eid = ebase + j * _CHUNK + g * _L + lane
                ex = jnp.where(eid < n_edges, jnp.exp(acc), 0.0)
                ex_v[pl.ds(g * _L, _L)] = ex

            @pl.when(j + 1 < cpw)
            def _():
                issue_sig_gathers(j + 1)

            pltpu.make_async_copy(msgh_hbm.at[midx_v.at[j]], msgb[b],
                                  gc[b]).wait()

            def scale_row(r, _):
                spl = plsc.load_gather(ex_v, [jnp.full((_L,), r, jnp.int32)])
                for q in range(_MH // _L):
                    scaled0[r, pl.ds(q * _L, _L)] = (
                        msgb[b][r, pl.ds(q * _L, _L)] * spl)
                scaled0[r, pl.ds(_MH, _L)] = jnp.where(
                    lane == 0, spl, jnp.zeros((_L,), jnp.float32))
                return 0
            lax.fori_loop(0, _CHUNK, scale_row, 0, unroll=4)

            @pl.when(j + 2 < cpw)
            def _():
                issue_msg_gather(j + 2, b)

            pltpu.sync_copy(scaled0, p_sh.at[src_v.at[j]], add=True)

        def chunk_pair(j0, _):
            do_chunk(j0, 0)
            do_chunk(j0 + 1, 1)
            return 0

        lax.fori_loop(0, cpw // 2, lambda t, c: chunk_pair(2 * t, c), 0,
                      unroll=False)
        plsc.subcore_barrier()

        for k in range((nblocks + _NS - 1) // _NS):
            b = sid + _NS * k
            @pl.when(b < nblocks)
            def _():
                pltpu.sync_copy(
                    p_sh.at[pl.ds(b * zrows, zrows)],
                    out_hbm.at[cid].at[pl.ds(b * zrows, zrows)])

    return pl.kernel(
        body,
        out_type=jax.ShapeDtypeStruct((_NC, n_nodes, _DW), jnp.float32),
        mesh=mesh,
        scratch_types=[
            pltpu.VMEM((cpw, _CHUNK), jnp.int32),
            pltpu.VMEM((cpw, _CHUNK), jnp.int32),
            pltpu.VMEM((cpw, _CHUNK), jnp.int32),
            pltpu.VMEM((_CHUNK, d_sig), jnp.float32),
            pltpu.VMEM((_CHUNK, d_sig), jnp.float32),
            pltpu.VMEM((_CHUNK, _MH), jnp.float32),
            pltpu.VMEM((_CHUNK, _MH), jnp.float32),
            pltpu.VMEM((_CHUNK, _DW), jnp.float32),
            pltpu.VMEM((_CHUNK,), jnp.float32),
            pltpu.VMEM((200, _DW), jnp.float32),
            pltpu.VMEM_SHARED((n_nodes, _DW), jnp.float32),
        ] + [pltpu.SemaphoreType.DMA] * 4,
        compiler_params=pltpu.CompilerParams(
            needs_layout_passes=False, use_tc_tiling_on_sc=False),
    )


def _combine_body(h_ref, p0_ref, p1_ref, wh_ref, wc0_ref, wc1_ref, ba_ref,
                  wd_ref, bd_ref, out_ref):
    p0 = p0_ref[...]
    p1 = p1_ref[...]
    comm0 = p0[:, :_MH] / jnp.maximum(p0[:, _MH:_MH + 1], 1e-30)
    comm1 = p1[:, :_MH] / jnp.maximum(p1[:, _MH:_MH + 1], 1e-30)
    combined = jnp.maximum(
        jnp.dot(h_ref[...], wh_ref[...], preferred_element_type=jnp.float32)
        + jnp.dot(comm0, wc0_ref[...], preferred_element_type=jnp.float32)
        + jnp.dot(comm1, wc1_ref[...], preferred_element_type=jnp.float32)
        + ba_ref[...], 0.0)
    out_ref[...] = (
        jnp.dot(combined, wd_ref[...], preferred_element_type=jnp.float32)
        + bd_ref[...])


def _combine(h, p0, p1, W_agg, b_agg, W_dec, b_dec, block_rows):
    n, d_h = h.shape
    d_out = W_dec.shape[1]
    grid = (n // block_rows,)
    full = lambda shape: pl.BlockSpec(shape, lambda i: (0, 0))
    return pl.pallas_call(
        _combine_body,
        grid=grid,
        in_specs=[
            pl.BlockSpec((block_rows, d_h), lambda i: (i, 0)),
            pl.BlockSpec((block_rows, _DW), lambda i: (i, 0)),
            pl.BlockSpec((block_rows, _DW), lambda i: (i, 0)),
            full((d_h, d_h)), full((_MH, d_h)), full((_MH, d_h)),
            full((1, d_h)),
            full((d_h, d_out)), full((1, d_out)),
        ],
        out_specs=pl.BlockSpec((block_rows, d_out), lambda i: (i, 0)),
        out_shape=jax.ShapeDtypeStruct((n, d_out), jnp.float32),
    )(h, p0, p1, W_agg[:d_h], W_agg[d_h:d_h + _MH], W_agg[d_h + _MH:],
      b_agg.reshape(1, -1), W_dec, b_dec.reshape(1, -1))


@jax.jit
def kernel(x, edge_index, W_enc, b_enc, W_sig, b_sig, W_msg, b_msg,
           W_agg, b_agg, W_dec, b_dec):
    n = x.shape[0]
    e = edge_index.shape[1]
    d_sig = W_sig.shape[1]

    h, sig, msg0, msg1 = _encode(x, W_enc, b_enc, W_sig, b_sig, W_msg, b_msg,
                                 block_rows=400)
    msgh = jnp.concatenate([msg0, msg1], axis=0)

    slab = _CHUNK * _NS
    cpw = (e + slab - 1) // slab
    cpw += cpw % 2
    e_pad = cpw * slab
    src = edge_index[0].astype(jnp.int32)
    dst = edge_index[1].astype(jnp.int32)
    pad = e_pad - e
    if pad:
        src = jnp.concatenate([src, jnp.zeros((pad,), jnp.int32)])
        dst = jnp.concatenate([dst, jnp.zeros((pad,), jnp.int32)])
    src3 = src.reshape(_NS, cpw, _CHUNK)
    dst3 = dst.reshape(_NS, cpw, _CHUNK)

    p = _sc_edge_kernel(n, e, cpw, d_sig)(sig, msgh, src3, dst3)

    return _combine(h, p[0], p[1], W_agg, b_agg, W_dec, b_dec, block_rows=400)
